# baseline (speedup 1.0000x reference)
import functools

import jax
import jax.numpy as jnp
from jax.experimental import pallas as pl
from jax.experimental.pallas import tpu as pltpu

D_MODEL = 768
N_EXPERTS = 64
K = 2
NUM_GATES = 4
NG = NUM_GATES * N_EXPERTS

BT = 4096


def _router_kernel(x1_ref, x2_ref, w_ref,
                   idx_ref, scr_ref, probs_ref, imp_ref, load_ref):
    logits = jax.lax.dot_general(
        x1_ref[0], w_ref[:D_MODEL // 2],
        dimension_numbers=(((1,), (0,)), ((), ())),
        preferred_element_type=jnp.float32,
    ) + jax.lax.dot_general(
        x2_ref[0], w_ref[D_MODEL // 2:],
        dimension_numbers=(((1,), (0,)), ((), ())),
        preferred_element_type=jnp.float32,
    )
    lt = logits.T
    probs_t = None
    for g in range(NUM_GATES):
        lg = lt[g * N_EXPERTS:(g + 1) * N_EXPERTS, :]
        mg = jnp.max(lg, axis=0, keepdims=True)
        eg = jnp.exp(lg - mg)
        sg = jnp.sum(eg, axis=0, keepdims=True)
        pg = eg / sg
        probs_t = pg if probs_t is None else probs_t + pg
    probs_t = probs_t * (1.0 / NUM_GATES)
    probs_ref[0] = probs_t.T

    iota = jax.lax.broadcasted_iota(
        jnp.int32, (N_EXPERTS, BT), 0).astype(jnp.float32)
    m1 = jnp.max(probs_t, axis=0, keepdims=True)
    i1 = jnp.min(jnp.where(probs_t == m1, iota, float(N_EXPERTS)),
                 axis=0, keepdims=True)
    masked = jnp.where(iota == i1, -jnp.inf, probs_t)
    m2 = jnp.max(masked, axis=0, keepdims=True)
    i2 = jnp.min(jnp.where(masked == m2, iota, float(N_EXPERTS)),
                 axis=0, keepdims=True)
    den = jnp.maximum(m1 + m2, 1e-9)
    idx_ref[0] = jnp.concatenate([i1, i2], axis=0).T.astype(jnp.int32)
    scr_ref[0] = jnp.concatenate([m1 / den, m2 / den], axis=0).T

    psum = jnp.sum(probs_t, axis=1, keepdims=True)
    lsum = jnp.sum((probs_t > 0.0).astype(jnp.float32), axis=1, keepdims=True)
    imp_ref[0] = jnp.broadcast_to(psum.T, imp_ref.shape[1:])
    load_ref[0] = jnp.broadcast_to(lsum.T, load_ref.shape[1:])


@functools.partial(jax.jit, static_argnames=())
def kernel(x, W):
    B, S, D = x.shape
    T = B * S
    wt = W.reshape(NG, D).T

    grid = (B, S // BT)
    out = pl.pallas_call(
        _router_kernel,
        grid=grid,
        in_specs=[
            pl.BlockSpec((1, BT, D // 2), lambda b, i: (b, i, 0)),
            pl.BlockSpec((1, BT, D // 2), lambda b, i: (b, i, 1)),
            pl.BlockSpec((D, NG), lambda b, i: (0, 0)),
        ],
        out_specs=[
            pl.BlockSpec((1, BT, K), lambda b, i: (b, i, 0)),
            pl.BlockSpec((1, BT, K), lambda b, i: (b, i, 0)),
            pl.BlockSpec((1, BT, N_EXPERTS), lambda b, i: (b, i, 0)),
            pl.BlockSpec((1, 8, N_EXPERTS),
                         lambda b, i: (b * (S // BT) + i, 0, 0)),
            pl.BlockSpec((1, 8, N_EXPERTS),
                         lambda b, i: (b * (S // BT) + i, 0, 0)),
        ],
        out_shape=[
            jax.ShapeDtypeStruct((B, S, K), jnp.int32),
            jax.ShapeDtypeStruct((B, S, K), jnp.float32),
            jax.ShapeDtypeStruct((B, S, N_EXPERTS), jnp.float32),
            jax.ShapeDtypeStruct((T // BT, 8, N_EXPERTS), jnp.float32),
            jax.ShapeDtypeStruct((T // BT, 8, N_EXPERTS), jnp.float32),
        ],
        compiler_params=pltpu.CompilerParams(
            dimension_semantics=("parallel", "parallel")),
    )(x, x, wt)
    idx, scores, probs_full, imp_acc, load_acc = out
    inv_t = 1.0 / float(T)
    importance = jnp.sum(imp_acc[:, 0], axis=0) * inv_t
    load = jnp.sum(load_acc[:, 0], axis=0) * inv_t
    return (idx, scores, probs_full, importance, load)


# speedup vs baseline: 1.0373x; 1.0373x over previous
import functools

import jax
import jax.numpy as jnp
from jax.experimental import pallas as pl
from jax.experimental.pallas import tpu as pltpu

D_MODEL = 768
N_EXPERTS = 64
K = 2
NUM_GATES = 4
NG = NUM_GATES * N_EXPERTS

BT = 4096


def _router_kernel(x1_ref, x2_ref, w_ref,
                   idx_ref, scr_ref, probs_ref, imp_ref, load_ref):
    logits = jnp.concatenate([
        jax.lax.dot_general(
            x1_ref[0], w_ref[:],
            dimension_numbers=(((1,), (0,)), ((), ())),
            preferred_element_type=jnp.float32,
        ),
        jax.lax.dot_general(
            x2_ref[0], w_ref[:],
            dimension_numbers=(((1,), (0,)), ((), ())),
            preferred_element_type=jnp.float32,
        )], axis=0)
    lt = logits.T
    probs_t = None
    for g in range(NUM_GATES):
        lg = lt[g * N_EXPERTS:(g + 1) * N_EXPERTS, :]
        mg = jnp.max(lg, axis=0, keepdims=True)
        eg = jnp.exp(lg - mg)
        sg = jnp.sum(eg, axis=0, keepdims=True)
        pg = eg / sg
        probs_t = pg if probs_t is None else probs_t + pg
    probs_t = probs_t * (1.0 / NUM_GATES)
    probs_ref[0] = probs_t.T

    iota = jax.lax.broadcasted_iota(
        jnp.int32, (N_EXPERTS, BT), 0).astype(jnp.float32)
    m1 = jnp.max(probs_t, axis=0, keepdims=True)
    i1 = jnp.min(jnp.where(probs_t == m1, iota, float(N_EXPERTS)),
                 axis=0, keepdims=True)
    masked = jnp.where(iota == i1, -jnp.inf, probs_t)
    m2 = jnp.max(masked, axis=0, keepdims=True)
    i2 = jnp.min(jnp.where(masked == m2, iota, float(N_EXPERTS)),
                 axis=0, keepdims=True)
    den = jnp.maximum(m1 + m2, 1e-9)
    idx_ref[0] = jnp.concatenate([i1, i2], axis=0).T.astype(jnp.int32)
    scr_ref[0] = jnp.concatenate([m1 / den, m2 / den], axis=0).T

    psum = jnp.sum(probs_t, axis=1, keepdims=True)
    lsum = jnp.sum((probs_t > 0.0).astype(jnp.float32), axis=1, keepdims=True)
    imp_ref[0] = jnp.broadcast_to(psum.T, imp_ref.shape[1:])
    load_ref[0] = jnp.broadcast_to(lsum.T, load_ref.shape[1:])


@functools.partial(jax.jit, static_argnames=())
def kernel(x, W):
    B, S, D = x.shape
    T = B * S
    wt = W.reshape(NG, D).T

    grid = (B, S // BT)
    out = pl.pallas_call(
        _router_kernel,
        grid=grid,
        in_specs=[
            pl.BlockSpec((1, BT // 2, D), lambda b, i: (b, 2 * i, 0)),
            pl.BlockSpec((1, BT // 2, D), lambda b, i: (b, 2 * i + 1, 0)),
            pl.BlockSpec((D, NG), lambda b, i: (0, 0)),
        ],
        out_specs=[
            pl.BlockSpec((1, BT, K), lambda b, i: (b, i, 0)),
            pl.BlockSpec((1, BT, K), lambda b, i: (b, i, 0)),
            pl.BlockSpec((1, BT, N_EXPERTS), lambda b, i: (b, i, 0)),
            pl.BlockSpec((1, 8, N_EXPERTS),
                         lambda b, i: (b * (S // BT) + i, 0, 0)),
            pl.BlockSpec((1, 8, N_EXPERTS),
                         lambda b, i: (b * (S // BT) + i, 0, 0)),
        ],
        out_shape=[
            jax.ShapeDtypeStruct((B, S, K), jnp.int32),
            jax.ShapeDtypeStruct((B, S, K), jnp.float32),
            jax.ShapeDtypeStruct((B, S, N_EXPERTS), jnp.float32),
            jax.ShapeDtypeStruct((T // BT, 8, N_EXPERTS), jnp.float32),
            jax.ShapeDtypeStruct((T // BT, 8, N_EXPERTS), jnp.float32),
        ],
        compiler_params=pltpu.CompilerParams(
            dimension_semantics=("parallel", "parallel")),
    )(x, x, wt)
    idx, scores, probs_full, imp_acc, load_acc = out
    inv_t = 1.0 / float(T)
    importance = jnp.sum(imp_acc[:, 0], axis=0) * inv_t
    load = jnp.sum(load_acc[:, 0], axis=0) * inv_t
    return (idx, scores, probs_full, importance, load)


# speedup vs baseline: 1.0710x; 1.0325x over previous
import functools

import jax
import jax.numpy as jnp
from jax.experimental import pallas as pl
from jax.experimental.pallas import tpu as pltpu

D_MODEL = 768
N_EXPERTS = 64
K = 2
NUM_GATES = 4
NG = NUM_GATES * N_EXPERTS

BT = 4096


def _router_kernel(x_ref, w_ref,
                   idx_ref, scr_ref, probs_ref, imp_ref, load_ref):
    logits = jax.lax.dot_general(
        x_ref[0], w_ref[:],
        dimension_numbers=(((1,), (0,)), ((), ())),
        preferred_element_type=jnp.float32,
    )
    lt = logits.T
    probs_t = None
    for g in range(NUM_GATES):
        lg = lt[g * N_EXPERTS:(g + 1) * N_EXPERTS, :]
        mg = jnp.max(lg, axis=0, keepdims=True)
        eg = jnp.exp(lg - mg)
        sg = jnp.sum(eg, axis=0, keepdims=True)
        pg = eg / sg
        probs_t = pg if probs_t is None else probs_t + pg
    probs_t = probs_t * (1.0 / NUM_GATES)
    probs_ref[0] = probs_t.T

    iota = jax.lax.broadcasted_iota(
        jnp.int32, (N_EXPERTS, BT), 0).astype(jnp.float32)
    m1 = jnp.max(probs_t, axis=0, keepdims=True)
    i1 = jnp.min(jnp.where(probs_t == m1, iota, float(N_EXPERTS)),
                 axis=0, keepdims=True)
    masked = jnp.where(iota == i1, -jnp.inf, probs_t)
    m2 = jnp.max(masked, axis=0, keepdims=True)
    i2 = jnp.min(jnp.where(masked == m2, iota, float(N_EXPERTS)),
                 axis=0, keepdims=True)
    den = jnp.maximum(m1 + m2, 1e-9)
    idx_ref[0] = jnp.concatenate([i1, i2], axis=0).T.astype(jnp.int32)
    scr_ref[0] = jnp.concatenate([m1 / den, m2 / den], axis=0).T

    psum = jnp.sum(probs_t, axis=1, keepdims=True)
    lsum = jnp.sum((probs_t > 0.0).astype(jnp.float32), axis=1, keepdims=True)
    imp_ref[0] = jnp.broadcast_to(psum.T, imp_ref.shape[1:])
    load_ref[0] = jnp.broadcast_to(lsum.T, load_ref.shape[1:])


@functools.partial(jax.jit, static_argnames=())
def kernel(x, W):
    B, S, D = x.shape
    T = B * S
    wt = W.reshape(NG, D).T

    grid = (B, S // BT)
    out = pl.pallas_call(
        _router_kernel,
        grid=grid,
        in_specs=[
            pl.BlockSpec((1, BT, D), lambda b, i: (b, i, 0)),
            pl.BlockSpec((D, NG), lambda b, i: (0, 0)),
        ],
        out_specs=[
            pl.BlockSpec((1, BT, K), lambda b, i: (b, i, 0)),
            pl.BlockSpec((1, BT, K), lambda b, i: (b, i, 0)),
            pl.BlockSpec((1, BT, N_EXPERTS), lambda b, i: (b, i, 0)),
            pl.BlockSpec((1, 8, N_EXPERTS),
                         lambda b, i: (b * (S // BT) + i, 0, 0)),
            pl.BlockSpec((1, 8, N_EXPERTS),
                         lambda b, i: (b * (S // BT) + i, 0, 0)),
        ],
        out_shape=[
            jax.ShapeDtypeStruct((B, S, K), jnp.int32),
            jax.ShapeDtypeStruct((B, S, K), jnp.float32),
            jax.ShapeDtypeStruct((B, S, N_EXPERTS), jnp.float32),
            jax.ShapeDtypeStruct((T // BT, 8, N_EXPERTS), jnp.float32),
            jax.ShapeDtypeStruct((T // BT, 8, N_EXPERTS), jnp.float32),
        ],
        compiler_params=pltpu.CompilerParams(
            dimension_semantics=("parallel", "parallel")),
    )(x, wt)
    idx, scores, probs_full, imp_acc, load_acc = out
    inv_t = 1.0 / float(T)
    importance = jnp.sum(imp_acc[:, 0], axis=0) * inv_t
    load = jnp.sum(load_acc[:, 0], axis=0) * inv_t
    return (idx, scores, probs_full, importance, load)
